# baseline (speedup 1.0000x reference)
import functools

import jax
import jax.numpy as jnp
from jax import lax
from jax.experimental import pallas as pl
from jax.experimental.pallas import tpu as pltpu
from jax.experimental.pallas import tpu_sc as plsc

B, N = 64, 8192
K_STATIC = 4096
L = 16
NVEC = N // L
NW = 32
ROWS_PER_W = B // NW
HIST = 1024
HVEC = HIST // L
NH = 4


def _bucket_addr(digit):
    return jnp.bitwise_or(lax.shift_left(jnp.bitwise_and(digit, 63), 4),
                          lax.shift_right_logical(digit, 6))


def _bits(row_v, i):
    return plsc.bitcast(row_v[pl.ds(i * L, L)], jnp.int32)


def _level(row_v, hists, sh, pfx, n_sub, r):
    ones = jnp.ones((L,), jnp.int32)
    zeros = jnp.zeros((L,), jnp.int32)
    iota = lax.iota(jnp.int32, L)
    thresh = n_sub - r

    @plsc.parallel_loop(0, NVEC // NH, unroll=2)
    def _(i):
        for j in range(NH):
            v = _bits(row_v, i * NH + j)
            cond = lax.shift_right_logical(v, sh + 10) == pfx
            digit = jnp.bitwise_and(lax.shift_right_logical(v, sh),
                                    jnp.int32(HIST - 1))
            plsc.addupdate_scatter(hists[j], [_bucket_addr(digit)], ones,
                                   mask=cond)

    @plsc.parallel_loop(0, HVEC, unroll=4, carry=zeros)
    def acc(c, acc_in):
        s = zeros
        for j in range(NH):
            s = s + hists[j][pl.ds(c * L, L)]
        return acc_in + s

    coarse = plsc.cumsum(acc)
    ind_c = coarse <= thresh
    l_star = plsc.all_reduce_population_count(ind_c)
    pbefore = jnp.sum(jnp.where(ind_c, acc, 0))
    rel = thresh - pbefore

    col_base = iota * L + l_star
    cols = []
    for g in range(4):
        col = zeros
        for j in range(NH):
            col = col + plsc.load_gather(hists[j],
                                         [col_base + jnp.int32(256 * g)])
        cols.append(col)
    csum = [jnp.sum(c) for c in cols]
    g_star = jnp.int32(0)
    before_g = jnp.int32(0)
    run = jnp.int32(0)
    for g in range(4):
        nrun = run + csum[g]
        take = nrun <= rel
        g_star = g_star + jnp.where(take, 1, 0).astype(jnp.int32)
        before_g = before_g + jnp.where(take, csum[g], 0)
        run = nrun
    col_star = cols[3]
    for g in range(3):
        col_star = jnp.where(jnp.full((L,), g_star == g, jnp.bool_),
                             cols[g], col_star)
    cc = plsc.cumsum(col_star) + before_g
    ind = cc <= rel
    dwithin = plsc.all_reduce_population_count(ind)
    a_dm1 = pbefore + before_g + jnp.sum(jnp.where(ind, col_star, 0))
    big = jnp.int32(2**30)
    a_d = pbefore + jnp.min(jnp.where(ind, big, cc))
    d = l_star * 64 + g_star * 16 + dwithin

    @plsc.parallel_loop(0, HVEC, unroll=4)
    def _(c):
        for j in range(NH):
            hists[j][pl.ds(c * L, L)] = zeros

    return d, a_dm1, a_d - a_dm1


def _row_topk_mask(row_v, hists, out_v, out8_v, k):
    zeros = jnp.zeros((L,), jnp.int32)
    iota4 = lax.iota(jnp.int32, L) * 4

    def level_body(lvl, carry):
        n_sub, r, prefix = carry
        sh = 20 - 10 * lvl
        d, a_dm1, n_d = _level(row_v, hists, sh, prefix, n_sub, r)
        r = r - (n_sub - (a_dm1 + n_d))
        prefix = jnp.bitwise_or(lax.shift_left(prefix, 10), d)
        return n_d, r, prefix

    n_sub, r, t_bits = lax.fori_loop(
        0, 3, level_body, (jnp.int32(N), jnp.int32(k), zeros))

    @plsc.parallel_loop(0, NVEC // 4, unroll=2)
    def _(c):
        word = zeros
        for j in range(4):
            g = plsc.load_gather(row_v, [iota4 + (c * 64 + j)])
            v = plsc.bitcast(g, jnp.int32)
            word = jnp.bitwise_or(
                word, jnp.where(v >= t_bits, jnp.int32(1 << (8 * j)),
                                jnp.int32(0)))
        out8_v[pl.ds(c * L, L)] = word

    n_fix = jnp.where(r == n_sub, 0, NVEC)

    def exact_body(i, eqc):
        v = _bits(row_v, i)
        gt = v > t_bits
        eq = v == t_bits
        cum = plsc.cumsum(jnp.where(eq, 1, 0).astype(jnp.int32)) + eqc
        sel = jnp.logical_and(eq, cum <= r)
        out_v[pl.ds(i * L, L)] = jnp.where(jnp.logical_or(gt, sel), 1, 0
                                           ).astype(jnp.int32)
        return eqc + plsc.all_reduce_population_count(eq)

    lax.fori_loop(0, n_fix, exact_body, zeros)

    def repack_body(c, _):
        word = zeros
        for j in range(4):
            m = plsc.load_gather(out_v, [iota4 + (c * 64 + j)])
            word = jnp.bitwise_or(word, lax.shift_left(m, jnp.int32(8 * j)))
        out8_v[pl.ds(c * L, L)] = word
        return 0

    lax.fori_loop(0, n_fix // 4, repack_body, 0)


def _make_sc_kernel(k):
    mesh = plsc.VectorSubcoreMesh(core_axis_name="c", subcore_axis_name="s")

    @functools.partial(
        pl.kernel,
        out_type=jax.ShapeDtypeStruct((B, N // 4), jnp.int32),
        mesh=mesh,
        compiler_params=pltpu.CompilerParams(needs_layout_passes=False),
        scratch_types=[
            pltpu.VMEM((N,), jnp.float32),
            pltpu.VMEM((N,), jnp.int32),
            pltpu.VMEM((N // 4,), jnp.int32),
        ] + [pltpu.VMEM((HIST,), jnp.int32) for _ in range(NH)],
    )
    def sc_topk_mask(scores_hbm, out_hbm, row_v, out_v, out8_v, *hists):
        wid = lax.axis_index("s") * 2 + lax.axis_index("c")
        zeros = jnp.zeros((L,), jnp.int32)

        @plsc.parallel_loop(0, HVEC, unroll=4)
        def _(c):
            for j in range(NH):
                hists[j][pl.ds(c * L, L)] = zeros

        def row_body(rr, _):
            row = wid * ROWS_PER_W + rr
            pltpu.sync_copy(scores_hbm.at[row], row_v)
            _row_topk_mask(row_v, hists, out_v, out8_v, k)
            pltpu.sync_copy(out8_v, out_hbm.at[row])
            return 0

        lax.fori_loop(0, ROWS_PER_W, row_body, 0)

    return sc_topk_mask


def kernel(scores, k):
    del k
    packed = _make_sc_kernel(K_STATIC)(scores)
    mask_i8 = lax.bitcast_convert_type(packed, jnp.int8)
    return mask_i8.reshape(B, N).astype(bool)


# speedup vs baseline: 1.1375x; 1.1375x over previous
import functools

import jax
import jax.numpy as jnp
from jax import lax
from jax.experimental import pallas as pl
from jax.experimental.pallas import tpu as pltpu
from jax.experimental.pallas import tpu_sc as plsc

B, N = 64, 8192
K_STATIC = 4096
L = 16
NVEC = N // L
NW = 32
ROWS_PER_W = B // NW
HIST = 1024
HVEC = HIST // L
NH = 4


def _bucket_addr(digit):
    return jnp.bitwise_or(lax.shift_left(jnp.bitwise_and(digit, 63), 4),
                          lax.shift_right_logical(digit, 6))


def _bits(row_v, i):
    return plsc.bitcast(row_v[pl.ds(i * L, L)], jnp.int32)


def _level(row_v, hists, sh, pfx, n_sub, r):
    ones = jnp.ones((L,), jnp.int32)
    zeros = jnp.zeros((L,), jnp.int32)
    iota = lax.iota(jnp.int32, L)
    thresh = n_sub - r

    @plsc.parallel_loop(0, NVEC // NH, unroll=2)
    def _(i):
        for j in range(NH):
            v = _bits(row_v, i * NH + j)
            cond = lax.shift_right_logical(v, sh + 10) == pfx
            digit = jnp.bitwise_and(lax.shift_right_logical(v, sh),
                                    jnp.int32(HIST - 1))
            plsc.addupdate_scatter(hists[j], [_bucket_addr(digit)], ones,
                                   mask=cond)

    @plsc.parallel_loop(0, HVEC, unroll=4, carry=zeros)
    def acc(c, acc_in):
        s = zeros
        for j in range(NH):
            s = s + hists[j][pl.ds(c * L, L)]
        return acc_in + s

    coarse = plsc.cumsum(acc)
    ind_c = coarse <= thresh
    l_star = plsc.all_reduce_population_count(ind_c)
    pbefore = jnp.sum(jnp.where(ind_c, acc, 0))
    rel = thresh - pbefore

    col_base = iota * L + l_star
    cols = []
    for g in range(4):
        col = zeros
        for j in range(NH):
            col = col + plsc.load_gather(hists[j],
                                         [col_base + jnp.int32(256 * g)])
        cols.append(col)
    csum = [jnp.sum(c) for c in cols]
    g_star = jnp.int32(0)
    before_g = jnp.int32(0)
    run = jnp.int32(0)
    for g in range(4):
        nrun = run + csum[g]
        take = nrun <= rel
        g_star = g_star + jnp.where(take, 1, 0).astype(jnp.int32)
        before_g = before_g + jnp.where(take, csum[g], 0)
        run = nrun
    col_star = cols[3]
    for g in range(3):
        col_star = jnp.where(jnp.full((L,), g_star == g, jnp.bool_),
                             cols[g], col_star)
    cc = plsc.cumsum(col_star) + before_g
    ind = cc <= rel
    dwithin = plsc.all_reduce_population_count(ind)
    a_dm1 = pbefore + before_g + jnp.sum(jnp.where(ind, col_star, 0))
    big = jnp.int32(2**30)
    a_d = pbefore + jnp.min(jnp.where(ind, big, cc))
    d = l_star * 64 + g_star * 16 + dwithin

    @plsc.parallel_loop(0, HVEC, unroll=4)
    def _(c):
        for j in range(NH):
            hists[j][pl.ds(c * L, L)] = zeros

    return d, a_dm1, a_d - a_dm1


def _row_topk_mask(row_v, hists, out_v, k):
    zeros = jnp.zeros((L,), jnp.int32)

    def level_cond(carry):
        lvl, n_sub, r, prefix = carry
        return jnp.logical_and(lvl < 3, r != n_sub)

    def level_body(carry):
        lvl, n_sub, r, prefix = carry
        sh = 20 - 10 * lvl
        d, a_dm1, n_d = _level(row_v, hists, sh, prefix, n_sub, r)
        r = r - (n_sub - (a_dm1 + n_d))
        prefix = jnp.bitwise_or(lax.shift_left(prefix, 10), d)
        return lvl + 1, n_d, r, prefix

    lvl_done, n_sub, r, t_bits = lax.while_loop(
        level_cond, level_body,
        (jnp.int32(0), jnp.int32(N), jnp.int32(k), zeros))
    sh_fin = 30 - 10 * lvl_done

    @plsc.parallel_loop(0, NVEC, unroll=8)
    def _(i):
        v = lax.shift_right_logical(_bits(row_v, i), sh_fin)
        out_v[pl.ds(i * L, L)] = jnp.where(v >= t_bits, 1, 0).astype(jnp.int32)

    n_fix = jnp.where(r == n_sub, 0, NVEC)

    def exact_body(i, eqc):
        v = lax.shift_right_logical(_bits(row_v, i), sh_fin)
        gt = v > t_bits
        eq = v == t_bits
        cum = plsc.cumsum(jnp.where(eq, 1, 0).astype(jnp.int32)) + eqc
        sel = jnp.logical_and(eq, cum <= r)
        out_v[pl.ds(i * L, L)] = jnp.where(jnp.logical_or(gt, sel), 1, 0
                                           ).astype(jnp.int32)
        return eqc + plsc.all_reduce_population_count(eq)

    lax.fori_loop(0, n_fix, exact_body, zeros)


def _make_sc_kernel(k):
    mesh = plsc.VectorSubcoreMesh(core_axis_name="c", subcore_axis_name="s")

    @functools.partial(
        pl.kernel,
        out_type=jax.ShapeDtypeStruct((B, N), jnp.int32),
        mesh=mesh,
        compiler_params=pltpu.CompilerParams(needs_layout_passes=False),
        scratch_types=[
            pltpu.VMEM((N,), jnp.float32),
            pltpu.VMEM((N,), jnp.int32),
        ] + [pltpu.VMEM((HIST,), jnp.int32) for _ in range(NH)],
    )
    def sc_topk_mask(scores_hbm, out_hbm, row_v, out_v, *hists):
        wid = lax.axis_index("s") * 2 + lax.axis_index("c")
        zeros = jnp.zeros((L,), jnp.int32)

        @plsc.parallel_loop(0, HVEC, unroll=4)
        def _(c):
            for j in range(NH):
                hists[j][pl.ds(c * L, L)] = zeros

        def row_body(rr, _):
            row = wid * ROWS_PER_W + rr
            pltpu.sync_copy(scores_hbm.at[row], row_v)
            _row_topk_mask(row_v, hists, out_v, k)
            pltpu.sync_copy(out_v, out_hbm.at[row])
            return 0

        lax.fori_loop(0, ROWS_PER_W, row_body, 0)

    return sc_topk_mask


def kernel(scores, k):
    del k
    mask_i32 = _make_sc_kernel(K_STATIC)(scores)
    return mask_i32.astype(bool)


# speedup vs baseline: 1.3146x; 1.1557x over previous
import functools

import jax
import jax.numpy as jnp
from jax import lax
from jax.experimental import pallas as pl
from jax.experimental.pallas import tpu as pltpu
from jax.experimental.pallas import tpu_sc as plsc

B, N = 64, 8192
K_STATIC = 4096
L = 16
NVEC = N // L
NW = 32
ROWS_PER_W = B // NW
HIST = 1024
HVEC = HIST // L
NH = 4


def _bucket_addr(digit):
    return jnp.bitwise_or(lax.shift_left(jnp.bitwise_and(digit, 63), 4),
                          lax.shift_right_logical(digit, 6))


def _bits(row_v, i):
    return plsc.bitcast(row_v[pl.ds(i * L, L)], jnp.int32)


def _level(row_v, hists, sh, pfx, n_sub, r):
    ones = jnp.ones((L,), jnp.int32)
    zeros = jnp.zeros((L,), jnp.int32)
    iota = lax.iota(jnp.int32, L)
    thresh = n_sub - r

    @plsc.parallel_loop(0, NVEC // NH, unroll=2)
    def _(i):
        for j in range(NH):
            v = _bits(row_v, i * NH + j)
            cond = lax.shift_right_logical(v, sh + 10) == pfx
            digit = jnp.bitwise_and(lax.shift_right_logical(v, sh),
                                    jnp.int32(HIST - 1))
            plsc.addupdate_scatter(hists[j], [_bucket_addr(digit)], ones,
                                   mask=cond)

    @plsc.parallel_loop(0, HVEC, unroll=4, carry=zeros)
    def acc(c, acc_in):
        s = zeros
        for j in range(NH):
            s = s + hists[j][pl.ds(c * L, L)]
        return acc_in + s

    coarse = plsc.cumsum(acc)
    ind_c = coarse <= thresh
    l_star = plsc.all_reduce_population_count(ind_c)
    pbefore = jnp.sum(jnp.where(ind_c, acc, 0))
    rel = thresh - pbefore

    col_base = iota * L + l_star
    cols = []
    for g in range(4):
        col = zeros
        for j in range(NH):
            col = col + plsc.load_gather(hists[j],
                                         [col_base + jnp.int32(256 * g)])
        cols.append(col)
    csum = [jnp.sum(c) for c in cols]
    g_star = jnp.int32(0)
    before_g = jnp.int32(0)
    run = jnp.int32(0)
    for g in range(4):
        nrun = run + csum[g]
        take = nrun <= rel
        g_star = g_star + jnp.where(take, 1, 0).astype(jnp.int32)
        before_g = before_g + jnp.where(take, csum[g], 0)
        run = nrun
    col_star = cols[3]
    for g in range(3):
        col_star = jnp.where(jnp.full((L,), g_star == g, jnp.bool_),
                             cols[g], col_star)
    cc = plsc.cumsum(col_star) + before_g
    ind = cc <= rel
    dwithin = plsc.all_reduce_population_count(ind)
    a_dm1 = pbefore + before_g + jnp.sum(jnp.where(ind, col_star, 0))
    big = jnp.int32(2**30)
    a_d = pbefore + jnp.min(jnp.where(ind, big, cc))
    d = l_star * 64 + g_star * 16 + dwithin

    @plsc.parallel_loop(0, HVEC, unroll=4)
    def _(c):
        for j in range(NH):
            hists[j][pl.ds(c * L, L)] = zeros

    return d, a_dm1, a_d - a_dm1


def _row_topk_mask(row_v, hists, out_v, k):
    zeros = jnp.zeros((L,), jnp.int32)

    def level_cond(carry):
        lvl, n_sub, r, prefix = carry
        return jnp.logical_and(lvl < 3, r != n_sub)

    def level_body(carry):
        lvl, n_sub, r, prefix = carry
        sh = 20 - 10 * lvl
        d, a_dm1, n_d = _level(row_v, hists, sh, prefix, n_sub, r)
        r = r - (n_sub - (a_dm1 + n_d))
        prefix = jnp.bitwise_or(lax.shift_left(prefix, 10), d)
        return lvl + 1, n_d, r, prefix

    lvl_done, n_sub, r, t_bits = lax.while_loop(
        level_cond, level_body,
        (jnp.int32(0), jnp.int32(N), jnp.int32(k), zeros))
    sh_fin = 30 - 10 * lvl_done

    @plsc.parallel_loop(0, NVEC, unroll=8)
    def _(i):
        v = lax.shift_right_logical(_bits(row_v, i), sh_fin)
        out_v[pl.ds(i * L, L)] = jnp.where(v >= t_bits, 1, 0).astype(jnp.int32)

    n_fix = jnp.where(r == n_sub, 0, NVEC)

    def exact_body(i, eqc):
        v = lax.shift_right_logical(_bits(row_v, i), sh_fin)
        gt = v > t_bits
        eq = v == t_bits
        cum = plsc.cumsum(jnp.where(eq, 1, 0).astype(jnp.int32)) + eqc
        sel = jnp.logical_and(eq, cum <= r)
        out_v[pl.ds(i * L, L)] = jnp.where(jnp.logical_or(gt, sel), 1, 0
                                           ).astype(jnp.int32)
        return eqc + plsc.all_reduce_population_count(eq)

    lax.fori_loop(0, n_fix, exact_body, zeros)


def _make_sc_kernel(k, nrows):
    mesh = plsc.VectorSubcoreMesh(core_axis_name="c", subcore_axis_name="s")
    rows_per_w = nrows // NW

    @functools.partial(
        pl.kernel,
        out_type=jax.ShapeDtypeStruct((nrows, N), jnp.int32),
        mesh=mesh,
        compiler_params=pltpu.CompilerParams(needs_layout_passes=False),
        scratch_types=[
            pltpu.VMEM((N,), jnp.float32),
            pltpu.VMEM((N,), jnp.int32),
        ] + [pltpu.VMEM((HIST,), jnp.int32) for _ in range(NH)],
    )
    def sc_topk_mask(scores_hbm, out_hbm, row_v, out_v, *hists):
        wid = lax.axis_index("s") * 2 + lax.axis_index("c")
        zeros = jnp.zeros((L,), jnp.int32)

        @plsc.parallel_loop(0, HVEC, unroll=4)
        def _(c):
            for j in range(NH):
                hists[j][pl.ds(c * L, L)] = zeros

        def row_body(rr, _):
            row = wid * rows_per_w + rr
            pltpu.sync_copy(scores_hbm.at[row], row_v)
            _row_topk_mask(row_v, hists, out_v, k)
            pltpu.sync_copy(out_v, out_hbm.at[row])
            return 0

        lax.fori_loop(0, rows_per_w, row_body, 0)

    return sc_topk_mask


def _tc_topk_mask(scores, k):
    nr = scores.shape[0]

    def body(s_ref, o_ref):
        bits = lax.bitcast_convert_type(s_ref[...], jnp.int32)
        lo = jnp.zeros((nr, 1), jnp.int32)
        for step in range(30):
            t = lo + jnp.int32(1 << (29 - step))
            cnt = jnp.sum((bits >= t).astype(jnp.int32), axis=1,
                          keepdims=True)
            lo = jnp.where(cnt >= k, t, lo)
        t_bits = lo
        gt = bits > t_bits
        eq = bits == t_bits
        m = jnp.sum(gt.astype(jnp.int32), axis=1, keepdims=True)
        idx = lax.broadcasted_iota(jnp.int32, (nr, N), 1)
        jsel = jnp.zeros((nr, 1), jnp.int32)
        for step in range(14):
            jt = jsel + jnp.int32(1 << (13 - step))
            cnt = m + jnp.sum(
                jnp.logical_and(eq, idx < jt).astype(jnp.int32), axis=1,
                keepdims=True)
            jsel = jnp.where(cnt <= k, jt, jsel)
        mask = jnp.logical_or(gt, jnp.logical_and(eq, idx < jsel))
        o_ref[...] = mask

    return pl.pallas_call(
        body,
        out_shape=jax.ShapeDtypeStruct((nr, N), jnp.bool_),
    )(scores)


SC_ROWS = 32


def kernel(scores, k):
    del k
    sc_mask = _make_sc_kernel(K_STATIC, SC_ROWS)(scores[:SC_ROWS])
    tc_mask = _tc_topk_mask(scores[SC_ROWS:], K_STATIC)
    return jnp.concatenate([sc_mask.astype(bool), tc_mask], axis=0)


# speedup vs baseline: 1.4067x; 1.0701x over previous
import functools

import jax
import jax.numpy as jnp
from jax import lax
from jax.experimental import pallas as pl
from jax.experimental.pallas import tpu as pltpu
from jax.experimental.pallas import tpu_sc as plsc

B, N = 64, 8192
K_STATIC = 4096
L = 16
NVEC = N // L
NW = 32
ROWS_PER_W = B // NW
HIST = 1024
HVEC = HIST // L
NH = 4
SC_ROWS = 32


def _bucket_addr(digit):
    return jnp.bitwise_or(lax.shift_left(jnp.bitwise_and(digit, 63), 4),
                          lax.shift_right_logical(digit, 6))


def _bits(row_v, i):
    return plsc.bitcast(row_v[pl.ds(i * L, L)], jnp.int32)


def _level(row_v, hists, sh, pfx, n_sub, r):
    ones = jnp.ones((L,), jnp.int32)
    zeros = jnp.zeros((L,), jnp.int32)
    iota = lax.iota(jnp.int32, L)
    thresh = n_sub - r

    @plsc.parallel_loop(0, NVEC // NH, unroll=2)
    def _(i):
        for j in range(NH):
            v = _bits(row_v, i * NH + j)
            cond = lax.shift_right_logical(v, sh + 10) == pfx
            digit = jnp.bitwise_and(lax.shift_right_logical(v, sh),
                                    jnp.int32(HIST - 1))
            plsc.addupdate_scatter(hists[j], [_bucket_addr(digit)], ones,
                                   mask=cond)

    @plsc.parallel_loop(0, HVEC, unroll=4, carry=zeros)
    def acc(c, acc_in):
        s = zeros
        for j in range(NH):
            s = s + hists[j][pl.ds(c * L, L)]
        return acc_in + s

    coarse = plsc.cumsum(acc)
    ind_c = coarse <= thresh
    l_star = plsc.all_reduce_population_count(ind_c)
    pbefore = jnp.sum(jnp.where(ind_c, acc, 0))
    rel = thresh - pbefore

    col_base = iota * L + l_star
    cols = []
    for g in range(4):
        col = zeros
        for j in range(NH):
            col = col + plsc.load_gather(hists[j],
                                         [col_base + jnp.int32(256 * g)])
        cols.append(col)
    csum = [jnp.sum(c) for c in cols]
    g_star = jnp.int32(0)
    before_g = jnp.int32(0)
    run = jnp.int32(0)
    for g in range(4):
        nrun = run + csum[g]
        take = nrun <= rel
        g_star = g_star + jnp.where(take, 1, 0).astype(jnp.int32)
        before_g = before_g + jnp.where(take, csum[g], 0)
        run = nrun
    col_star = cols[3]
    for g in range(3):
        col_star = jnp.where(jnp.full((L,), g_star == g, jnp.bool_),
                             cols[g], col_star)
    cc = plsc.cumsum(col_star) + before_g
    ind = cc <= rel
    dwithin = plsc.all_reduce_population_count(ind)
    a_dm1 = pbefore + before_g + jnp.sum(jnp.where(ind, col_star, 0))
    big = jnp.int32(2**30)
    a_d = pbefore + jnp.min(jnp.where(ind, big, cc))
    d = l_star * 64 + g_star * 16 + dwithin

    @plsc.parallel_loop(0, HVEC, unroll=4)
    def _(c):
        for j in range(NH):
            hists[j][pl.ds(c * L, L)] = zeros

    return d, a_dm1, a_d - a_dm1


def _row_topk_mask(row_v, hists, out_v, k):
    zeros = jnp.zeros((L,), jnp.int32)

    def level_cond(carry):
        lvl, n_sub, r, prefix = carry
        return jnp.logical_and(lvl < 3, r != n_sub)

    def level_body(carry):
        lvl, n_sub, r, prefix = carry
        sh = 20 - 10 * lvl
        d, a_dm1, n_d = _level(row_v, hists, sh, prefix, n_sub, r)
        r = r - (n_sub - (a_dm1 + n_d))
        prefix = jnp.bitwise_or(lax.shift_left(prefix, 10), d)
        return lvl + 1, n_d, r, prefix

    lvl_done, n_sub, r, t_bits = lax.while_loop(
        level_cond, level_body,
        (jnp.int32(0), jnp.int32(N), jnp.int32(k), zeros))
    sh_fin = 30 - 10 * lvl_done

    @plsc.parallel_loop(0, NVEC, unroll=8)
    def _(i):
        v = lax.shift_right_logical(_bits(row_v, i), sh_fin)
        out_v[pl.ds(i * L, L)] = jnp.where(v >= t_bits, 1, 0).astype(jnp.int32)

    n_fix = jnp.where(r == n_sub, 0, NVEC)

    def exact_body(i, eqc):
        v = lax.shift_right_logical(_bits(row_v, i), sh_fin)
        gt = v > t_bits
        eq = v == t_bits
        cum = plsc.cumsum(jnp.where(eq, 1, 0).astype(jnp.int32)) + eqc
        sel = jnp.logical_and(eq, cum <= r)
        out_v[pl.ds(i * L, L)] = jnp.where(jnp.logical_or(gt, sel), 1, 0
                                           ).astype(jnp.int32)
        return eqc + plsc.all_reduce_population_count(eq)

    lax.fori_loop(0, n_fix, exact_body, zeros)


def _make_sc_kernel(k, nrows):
    mesh = plsc.VectorSubcoreMesh(core_axis_name="c", subcore_axis_name="s")
    rows_per_w = nrows // NW

    @functools.partial(
        pl.kernel,
        out_type=jax.ShapeDtypeStruct((nrows, N), jnp.int32),
        mesh=mesh,
        compiler_params=pltpu.CompilerParams(needs_layout_passes=False),
        scratch_types=[
            pltpu.VMEM((N,), jnp.float32),
            pltpu.VMEM((N,), jnp.int32),
        ] + [pltpu.VMEM((HIST,), jnp.int32) for _ in range(NH)],
    )
    def sc_topk_mask(scores_hbm, out_hbm, row_v, out_v, *hists):
        wid = lax.axis_index("s") * 2 + lax.axis_index("c")
        zeros = jnp.zeros((L,), jnp.int32)

        @plsc.parallel_loop(0, HVEC, unroll=4)
        def _(c):
            for j in range(NH):
                hists[j][pl.ds(c * L, L)] = zeros

        def row_body(rr, _):
            row = wid * rows_per_w + rr
            pltpu.sync_copy(scores_hbm.at[row], row_v)
            _row_topk_mask(row_v, hists, out_v, k)
            pltpu.sync_copy(out_v, out_hbm.at[row])
            return 0

        lax.fori_loop(0, rows_per_w, row_body, 0)

    return sc_topk_mask


def _tc_topk_mask(scores, k):
    nr = B - SC_ROWS

    def body(s_ref, o_ref):
        bits = lax.bitcast_convert_type(s_ref[...], jnp.int32)
        lo = jnp.zeros((nr, 1), jnp.int32)
        for step in range(30):
            t = lo + jnp.int32(1 << (29 - step))
            cnt = jnp.sum((bits >= t).astype(jnp.int32), axis=1,
                          keepdims=True)
            lo = jnp.where(cnt >= k, t, lo)
        t_bits = lo
        gt = bits > t_bits
        eq = bits == t_bits
        m = jnp.sum(gt.astype(jnp.int32), axis=1, keepdims=True)
        idx = lax.broadcasted_iota(jnp.int32, (nr, N), 1)
        jsel = jnp.zeros((nr, 1), jnp.int32)
        for step in range(14):
            jt = jsel + jnp.int32(1 << (13 - step))
            cnt = m + jnp.sum(
                jnp.logical_and(eq, idx < jt).astype(jnp.int32), axis=1,
                keepdims=True)
            jsel = jnp.where(cnt <= k, jt, jsel)
        mask = jnp.logical_or(gt, jnp.logical_and(eq, idx < jsel))
        o_ref[...] = mask

    return pl.pallas_call(
        body,
        grid=(1,),
        in_specs=[pl.BlockSpec((nr, N), lambda i: (1, 0))],
        out_specs=pl.BlockSpec((nr, N), lambda i: (0, 0)),
        out_shape=jax.ShapeDtypeStruct((nr, N), jnp.bool_),
    )(scores)


def kernel(scores, k):
    del k
    sc_mask = _make_sc_kernel(K_STATIC, SC_ROWS)(scores)
    tc_mask = _tc_topk_mask(scores, K_STATIC)
    return jnp.concatenate([sc_mask.astype(bool), tc_mask], axis=0)
